# Initial kernel scaffold; baseline (speedup 1.0000x reference)
import jax
import jax.numpy as jnp
from jax.experimental import pallas as pl


def kernel(x, edge_index, emb_table):
    raise NotImplementedError("write your pallas kernel here")


# speedup vs baseline: 3.3255x; 3.3255x over previous
import functools

import jax
import jax.numpy as jnp
from jax import lax
from jax.experimental import pallas as pl
from jax.experimental.pallas import tpu as pltpu
from jax.experimental.pallas import tpu_sc as plsc

_N_NODES = 10000
_ROW = 128
_NUM_TOK = 16
_TOK = 8
_EMB_ROWS = 256
_EMB_DIM = 8
_CH = 16
_NCHUNKS = _N_NODES // _CH
_CHW = _CH * _ROW
_NW = 32
_KMAX = -(-_NCHUNKS // _NW)

_mesh = plsc.VectorSubcoreMesh(core_axis_name="c", subcore_axis_name="s")


@functools.partial(
    pl.kernel,
    out_type=jax.ShapeDtypeStruct((_N_NODES * _ROW,), jnp.float32),
    mesh=_mesh,
    compiler_params=pltpu.CompilerParams(needs_layout_passes=False),
    scratch_types=[
        pltpu.VMEM((_CHW,), jnp.int32),
        pltpu.VMEM((_EMB_ROWS * _EMB_DIM,), jnp.float32),
        pltpu.VMEM((_CHW,), jnp.float32),
        pltpu.VMEM((_CH * _NUM_TOK,), jnp.int32),
    ],
)
def _node_emb(x_hbm, emb_hbm, out_hbm, xv, embv, outv, codesv):
    wid = lax.axis_index("s") * 2 + lax.axis_index("c")
    pltpu.sync_copy(emb_hbm, embv)

    lanes = lax.iota(jnp.int32, 16)
    col_base = lanes * _TOK
    epat = lanes & 7
    tok_base = lanes >> 3

    def chunk_body(k, carry):
        c = wid + _NW * k

        @pl.when(c < _NCHUNKS)
        def _():
            base = c * _CHW
            pltpu.sync_copy(x_hbm.at[pl.ds(base, _CHW)], xv)
            for n in range(_CH):
                nbase = col_base + n * _ROW
                codes = plsc.load_gather(xv, [nbase])
                for b in range(1, _TOK):
                    plane = plsc.load_gather(xv, [nbase + b])
                    codes = codes + (plane << b)
                codesv[pl.ds(n * _NUM_TOK, _NUM_TOK)] = codes
            for n in range(_CH):
                for v in range(_ROW // 16):
                    cpair = plsc.load_gather(
                        codesv, [tok_base + (n * _NUM_TOK + 2 * v)])
                    vals = plsc.load_gather(embv, [cpair * _EMB_DIM + epat])
                    outv[pl.ds(n * _ROW + v * 16, 16)] = vals
            pltpu.sync_copy(outv, out_hbm.at[pl.ds(base, _CHW)])

        return carry

    lax.fori_loop(0, _KMAX, chunk_body, 0)


def kernel(x, edge_index, emb_table):
    node_flat = _node_emb(
        x.reshape(-1).astype(jnp.int32), emb_table.reshape(-1))
    node_vec = node_flat.reshape(_N_NODES, _ROW)
    edge_vec = jnp.zeros((edge_index.shape[-1], _ROW), dtype=jnp.float32)
    return (node_vec, edge_index, edge_vec)
